# baseline (speedup 1.0000x reference)
import functools

import jax
from jax import lax
from jax.experimental import pallas as pl
from jax.experimental.pallas import tpu as pltpu
from jax.experimental.pallas import tpu_sc as plsc

_CH = 40
_NCH = 3


@functools.cache
def _sc_copy(b, s, d, dtype):
    info = plsc.get_sparse_core_info()
    nw = info.num_cores * info.num_subcores
    rows_per_w = s // nw
    tail = rows_per_w - _NCH * _CH
    mesh = plsc.VectorSubcoreMesh(core_axis_name="c", subcore_axis_name="s")

    @functools.partial(
        pl.kernel,
        mesh=mesh,
        out_type=jax.ShapeDtypeStruct((b, s, d), dtype),
        scratch_types=[
            pltpu.VMEM((_NCH, _CH, d), dtype),
            pltpu.SemaphoreType.DMA,
            pltpu.SemaphoreType.DMA,
        ],
    )
    def k(emb_hbm, out_hbm, buf, rsem, wsem):
        wid = lax.axis_index("s") * info.num_cores + lax.axis_index("c")
        base = wid * rows_per_w

        def rd(off, n, dst):
            return pltpu.async_copy(emb_hbm.at[pl.ds(base + off, n), :], dst, rsem)

        def wr(off, n, src):
            return [
                pltpu.async_copy(
                    src, out_hbm.at[bi, pl.ds(base + off, n), :], wsem
                )
                for bi in range(b)
            ]

        reads = [rd(c * _CH, _CH, buf.at[c]) for c in range(_NCH)]
        writes = {}
        for c in range(_NCH):
            reads[c].wait()
            writes[c] = wr(c * _CH, _CH, buf.at[c])
        if tail:
            for w in writes.pop(0):
                w.wait()
            tbuf = buf.at[0, pl.ds(0, tail), :]
            rd(_NCH * _CH, tail, tbuf).wait()
            writes[_NCH] = wr(_NCH * _CH, tail, tbuf)
        for c in sorted(writes):
            for w in writes[c]:
                w.wait()

    return k


def kernel(x, emb):
    b, s, d = x.shape
    return _sc_copy(b, s, d, emb.dtype)(emb)
